# baseline (speedup 1.0000x reference)
import functools

import jax
import jax.numpy as jnp
from jax import lax
from jax.experimental import pallas as pl
from jax.experimental.pallas import tpu as pltpu
from jax.experimental.pallas import tpu_sc as plsc

N = 10000
E = 320000
H = 128
G = 64
NEG = 0.2
EPS = 1e-30

NBLK = 2000
NW = 32
EC = E // NW
BB = 80
NBATCH = EC // BB
RS = 632
RSL = N - 15 * RS


def _node_tail(h, as_ref, ad_ref, asrc_ref, adst_ref, M_ref, mx_ref, i):
    a_s = jnp.sum(h * as_ref[...], axis=1, keepdims=True)
    a_d = jnp.sum(h * ad_ref[...], axis=1, keepdims=True)
    asrc_ref[...] = a_s
    adst_ref[...] = a_d

    @pl.when(i == 0)
    def _():
        mx_ref[0] = jnp.max(a_s)
        mx_ref[1] = jnp.max(a_d)

    mx_ref[0] = jnp.maximum(mx_ref[0], jnp.max(a_s))
    mx_ref[1] = jnp.maximum(mx_ref[1], jnp.max(a_d))

    @pl.when(i == pl.num_programs(0) - 1)
    def _():
        M_ref[0] = jnp.maximum(mx_ref[0] + mx_ref[1], 0.0)


def _tca_body(x_ref, W_ref, as_ref, ad_ref,
              h_ref, asrc_ref, adst_ref, M_ref, mx_ref):
    i = pl.program_id(0)
    h = jnp.dot(x_ref[...], W_ref[...], preferred_element_type=jnp.float32)
    h_ref[...] = h
    _node_tail(h, as_ref, ad_ref, asrc_ref, adst_ref, M_ref, mx_ref, i)


def _tcb_body(U_ref, den_ref, b_ref, W_ref, as_ref, ad_ref,
              h_ref, asrc_ref, adst_ref, M_ref, mx_ref):
    i = pl.program_id(0)
    u = U_ref[0] + U_ref[1]
    den = den_ref[0] + den_ref[1]
    out = jnp.maximum(u / (den + EPS) + b_ref[...], 0.0)
    h = jnp.dot(out, W_ref[...], preferred_element_type=jnp.float32)
    h_ref[...] = h
    _node_tail(h, as_ref, ad_ref, asrc_ref, adst_ref, M_ref, mx_ref, i)


def _tcc_body(U_ref, den_ref, b_ref, batch_ref, Wf_ref, bf_ref,
              y_ref, acc_ref):
    i = pl.program_id(0)
    u = U_ref[0] + U_ref[1]
    den = den_ref[0] + den_ref[1]
    out = jnp.maximum(u / (den + EPS) + b_ref[...], 0.0)
    gids = lax.broadcasted_iota(jnp.int32, (NBLK, G), 1)
    mask = (batch_ref[...] == gids).astype(jnp.float32)

    @pl.when(i == 0)
    def _():
        acc_ref[...] = jnp.zeros_like(acc_ref)

    acc_ref[...] += lax.dot_general(mask, out, (((0,), (0,)), ((), ())),
                                    preferred_element_type=jnp.float32)

    @pl.when(i == pl.num_programs(0) - 1)
    def _():
        y_ref[...] = jnp.dot(acc_ref[...], Wf_ref[...],
                             preferred_element_type=jnp.float32) + bf_ref[0]


def _tc_node_first(x, W, a_s, a_d):
    return pl.pallas_call(
        _tca_body,
        grid=(N // NBLK,),
        in_specs=[
            pl.BlockSpec((NBLK, H), lambda i: (i, 0)),
            pl.BlockSpec((H, H), lambda i: (0, 0)),
            pl.BlockSpec((1, H), lambda i: (0, 0)),
            pl.BlockSpec((1, H), lambda i: (0, 0)),
        ],
        out_specs=[
            pl.BlockSpec((NBLK, H), lambda i: (i, 0)),
            pl.BlockSpec((NBLK, 1), lambda i: (i, 0)),
            pl.BlockSpec((NBLK, 1), lambda i: (i, 0)),
            pl.BlockSpec(memory_space=pltpu.SMEM),
        ],
        out_shape=[
            jax.ShapeDtypeStruct((N, H), jnp.float32),
            jax.ShapeDtypeStruct((N, 1), jnp.float32),
            jax.ShapeDtypeStruct((N, 1), jnp.float32),
            jax.ShapeDtypeStruct((1,), jnp.float32),
        ],
        scratch_shapes=[pltpu.SMEM((2,), jnp.float32)],
    )(x, W, a_s.reshape(1, H), a_d.reshape(1, H))


def _tc_node_mid(U, den, b, W, a_s, a_d):
    return pl.pallas_call(
        _tcb_body,
        grid=(N // NBLK,),
        in_specs=[
            pl.BlockSpec((2, NBLK, H), lambda i: (0, i, 0)),
            pl.BlockSpec((2, NBLK, 1), lambda i: (0, i, 0)),
            pl.BlockSpec((1, H), lambda i: (0, 0)),
            pl.BlockSpec((H, H), lambda i: (0, 0)),
            pl.BlockSpec((1, H), lambda i: (0, 0)),
            pl.BlockSpec((1, H), lambda i: (0, 0)),
        ],
        out_specs=[
            pl.BlockSpec((NBLK, H), lambda i: (i, 0)),
            pl.BlockSpec((NBLK, 1), lambda i: (i, 0)),
            pl.BlockSpec((NBLK, 1), lambda i: (i, 0)),
            pl.BlockSpec(memory_space=pltpu.SMEM),
        ],
        out_shape=[
            jax.ShapeDtypeStruct((N, H), jnp.float32),
            jax.ShapeDtypeStruct((N, 1), jnp.float32),
            jax.ShapeDtypeStruct((N, 1), jnp.float32),
            jax.ShapeDtypeStruct((1,), jnp.float32),
        ],
        scratch_shapes=[pltpu.SMEM((2,), jnp.float32)],
    )(U, den.reshape(2, N, 1), b.reshape(1, H), W,
      a_s.reshape(1, H), a_d.reshape(1, H))


def _tc_pool(U, den, b, batch, Wf, bf):
    return pl.pallas_call(
        _tcc_body,
        grid=(N // NBLK,),
        in_specs=[
            pl.BlockSpec((2, NBLK, H), lambda i: (0, i, 0)),
            pl.BlockSpec((2, NBLK, 1), lambda i: (0, i, 0)),
            pl.BlockSpec((1, H), lambda i: (0, 0)),
            pl.BlockSpec((NBLK, 1), lambda i: (i, 0)),
            pl.BlockSpec((H, 1), lambda i: (0, 0)),
            pl.BlockSpec(memory_space=pltpu.SMEM),
        ],
        out_specs=pl.BlockSpec((G, 1), lambda i: (0, 0)),
        out_shape=jax.ShapeDtypeStruct((G, 1), jnp.float32),
        scratch_shapes=[pltpu.VMEM((G, H), jnp.float32)],
    )(U, den.reshape(2, N, 1), b.reshape(1, H),
      batch.reshape(N, 1), Wf, bf)


@functools.partial(
    pl.kernel,
    out_type=(jax.ShapeDtypeStruct((2, N, H), jnp.float32),
              jax.ShapeDtypeStruct((2, N), jnp.float32)),
    mesh=plsc.VectorSubcoreMesh(core_axis_name="c", subcore_axis_name="s"),
    compiler_params=pltpu.CompilerParams(needs_layout_passes=False,
                                         use_tc_tiling_on_sc=False),
    scratch_types=[
        pltpu.VMEM_SHARED((N, H), jnp.float32),
        pltpu.VMEM_SHARED((N,), jnp.float32),
        pltpu.VMEM((2 * BB,), jnp.int32),
        pltpu.VMEM((2 * BB,), jnp.int32),
        pltpu.VMEM((BB,), jnp.int32),
        pltpu.VMEM((BB,), jnp.int32),
        pltpu.VMEM((BB,), jnp.float32),
        pltpu.VMEM((BB,), jnp.float32),
        pltpu.VMEM((BB,), jnp.float32),
        pltpu.VMEM((BB,), jnp.float32),
        pltpu.VMEM((BB,), jnp.float32),
        pltpu.VMEM((BB,), jnp.float32),
        pltpu.VMEM((BB, H // 2), jnp.int32),
        pltpu.VMEM((BB, H // 2), jnp.int32),
        pltpu.VMEM((BB, H), jnp.float32),
        pltpu.VMEM((BB, H), jnp.float32),
        pltpu.VMEM((16,), jnp.float32),
        pltpu.SemaphoreType.DMA,
        pltpu.SemaphoreType.DMA,
        pltpu.SemaphoreType.DMA,
        pltpu.SemaphoreType.DMA,
        pltpu.SemaphoreType.DMA,
        pltpu.SemaphoreType.DMA,
        pltpu.SemaphoreType.DMA,
        pltpu.SemaphoreType.DMA,
    ],
)
def _sc_edge(h_hbm, asrc_hbm, adst_hbm, m_hbm,
             sd2_hbm, zr_hbm, zd_hbm,
             U_out, den_out,
             U_sh, den_sh,
             sdb0, sdb1, dstb0, dstb1, exb0, exb1, asb0, asb1, adb0, adb1,
             rbf0, rbf1, rf0, rf1, m_v,
             gs0, gs1, al0, al1, us0, us1, dn0, dn1):
    cid = lax.axis_index("c")
    sid = lax.axis_index("s")
    wid = sid * 2 + cid

    @pl.when(sid < 15)
    def _():
        off = pl.multiple_of(sid * RS, 8)
        pltpu.sync_copy(zr_hbm.at[pl.ds(off, RS)], U_sh.at[pl.ds(off, RS)])

    @pl.when(sid == 15)
    def _():
        pltpu.sync_copy(zr_hbm.at[pl.ds(15 * RS, RSL)],
                        U_sh.at[pl.ds(15 * RS, RSL)])

    @pl.when(sid == 0)
    def _():
        pltpu.sync_copy(zd_hbm, den_sh)

    pltpu.sync_copy(m_hbm, m_v)
    plsc.subcore_barrier()

    m = m_v[...]
    bufs = ((sdb0, dstb0, exb0, asb0, adb0, rbf0, rf0, gs0, al0, us0, dn0),
            (sdb1, dstb1, exb1, asb1, adb1, rbf1, rf1, gs1, al1, us1, dn1))

    def _prefetch(bn, buf, do_waits):
        sdb, dstb, exb, asb, adb, rbf, rf, gs, al, us, dn = buf

        @pl.when(do_waits)
        def _():
            pltpu.make_async_copy(rf, U_sh.at[dstb], us).wait()
            pltpu.make_async_copy(exb, den_sh.at[dstb], dn).wait()

        pltpu.sync_copy(sd2_hbm.at[wid * NBATCH + bn], sdb)
        for k in range(BB // 16):
            dstb[pl.ds(k * 16, 16)] = sdb[pl.ds(BB + k * 16, 16)]
        pltpu.async_copy(h_hbm.at[sdb.at[pl.ds(0, BB)]], rbf, gs)
        pltpu.async_copy(asrc_hbm.at[sdb.at[pl.ds(0, BB)]], asb, al)
        pltpu.async_copy(adst_hbm.at[dstb], adb, al)

    def _process(buf):
        sdb, dstb, exb, asb, adb, rbf, rf, gs, al, us, dn = buf
        pltpu.make_async_copy(asrc_hbm.at[sdb.at[pl.ds(0, BB)]], asb, al).wait()
        pltpu.make_async_copy(adst_hbm.at[dstb], adb, al).wait()

        @plsc.parallel_loop(0, BB // 16)
        def _(g):
            a = asb[pl.ds(g * 16, 16)] + adb[pl.ds(g * 16, 16)]
            a = jnp.maximum(a, a * NEG)
            exb[pl.ds(g * 16, 16)] = jnp.exp(a - m)

        pltpu.async_copy(exb, den_sh.at[dstb], dn, add=True)
        pltpu.make_async_copy(h_hbm.at[sdb.at[pl.ds(0, BB)]], rbf, gs).wait()

        @plsc.parallel_loop(0, BB, unroll=4)
        def _(e):
            eidx = jnp.full((16,), e, jnp.int32)
            ev = plsc.load_gather(exb, [eidx])
            for k in range(H // 32):
                x = rbf[e, pl.ds(k * 16, 16)]
                lo = plsc.bitcast(x << 16, jnp.float32) * ev
                hi = plsc.bitcast(x & jnp.int32(-65536), jnp.float32) * ev
                cbase = lax.iota(jnp.int32, 16) * 2 + (k * 32)
                plsc.store_scatter(rf, [eidx, cbase], lo)
                plsc.store_scatter(rf, [eidx, cbase + 1], hi)

        pltpu.async_copy(rf, U_sh.at[dstb], us, add=True)

    _prefetch(0, bufs[0], False)

    @pl.loop(0, NBATCH)
    def _(b):
        even = b % 2 == 0
        more = b + 1 < NBATCH

        @pl.when(jnp.logical_and(even, more))
        def _():
            _prefetch(b + 1, bufs[1], b >= 1)

        @pl.when(jnp.logical_and(jnp.logical_not(even), more))
        def _():
            _prefetch(b + 1, bufs[0], b >= 1)

        @pl.when(even)
        def _():
            _process(bufs[0])

        @pl.when(jnp.logical_not(even))
        def _():
            _process(bufs[1])

    pltpu.make_async_copy(rf0, U_sh.at[dstb0], us0).wait()
    pltpu.make_async_copy(exb0, den_sh.at[dstb0], dn0).wait()
    pltpu.make_async_copy(rf1, U_sh.at[dstb1], us1).wait()
    pltpu.make_async_copy(exb1, den_sh.at[dstb1], dn1).wait()

    plsc.subcore_barrier()

    @pl.when(sid < 15)
    def _():
        off = pl.multiple_of(sid * RS, 8)
        pltpu.sync_copy(U_sh.at[pl.ds(off, RS)],
                        U_out.at[cid, pl.ds(off, RS)])

    @pl.when(sid == 15)
    def _():
        pltpu.sync_copy(U_sh.at[pl.ds(15 * RS, RSL)],
                        U_out.at[cid, pl.ds(15 * RS, RSL)])

    @pl.when(sid == 0)
    def _():
        pltpu.sync_copy(den_sh, den_out.at[cid])


def kernel(x, edge_index, batch, dense_edge_idx, W1, a_src1, a_dst1, b1,
           W2, a_src2, a_dst2, b2, Wf, bf):
    src2 = edge_index[0].reshape(E // BB, BB)
    dst2 = edge_index[1].reshape(E // BB, BB)
    sd2 = jnp.concatenate([src2, dst2], axis=1)
    zr = jnp.zeros((N, H), jnp.float32)
    zd = jnp.zeros((N,), jnp.float32)

    def _pack(h):
        hb = h.astype(jnp.bfloat16).reshape(N, H // 2, 2)
        return lax.bitcast_convert_type(hb, jnp.int32)

    h1, asrc1, adst1, M1 = _tc_node_first(x, W1, a_src1, a_dst1)
    U1, den1 = _sc_edge(_pack(h1), asrc1.reshape(N), adst1.reshape(N),
                        jnp.broadcast_to(M1, (16,)), sd2, zr, zd)
    h2, asrc2, adst2, M2 = _tc_node_mid(U1, den1, b1, W2, a_src2, a_dst2)
    U2, den2 = _sc_edge(_pack(h2), asrc2.reshape(N), adst2.reshape(N),
                        jnp.broadcast_to(M2, (16,)), sd2, zr, zd)
    y = _tc_pool(U2, den2, b2, batch, Wf, bf.reshape(1))
    return y[:, 0]


# speedup vs baseline: 1.0306x; 1.0306x over previous
import functools

import jax
import jax.numpy as jnp
from jax import lax
from jax.experimental import pallas as pl
from jax.experimental.pallas import tpu as pltpu
from jax.experimental.pallas import tpu_sc as plsc

N = 10000
E = 320000
H = 128
G = 64
NEG = 0.2
EPS = 1e-30

NBLK = 2000
NW = 32
EC = E // NW
BB = 80
NBATCH = EC // BB
RS = 632
RSL = N - 15 * RS


def _node_tail(h, as_ref, ad_ref, asrc_ref, adst_ref, M_ref, mx_ref, i):
    a_s = jnp.sum(h * as_ref[...], axis=1, keepdims=True)
    a_d = jnp.sum(h * ad_ref[...], axis=1, keepdims=True)
    asrc_ref[...] = a_s
    adst_ref[...] = a_d

    @pl.when(i == 0)
    def _():
        mx_ref[0] = jnp.max(a_s)
        mx_ref[1] = jnp.max(a_d)

    mx_ref[0] = jnp.maximum(mx_ref[0], jnp.max(a_s))
    mx_ref[1] = jnp.maximum(mx_ref[1], jnp.max(a_d))

    @pl.when(i == pl.num_programs(0) - 1)
    def _():
        M_ref[0] = jnp.maximum(mx_ref[0] + mx_ref[1], 0.0)


def _tca_body(x_ref, W_ref, as_ref, ad_ref,
              h_ref, asrc_ref, adst_ref, M_ref, mx_ref):
    i = pl.program_id(0)
    h = jnp.dot(x_ref[...], W_ref[...], preferred_element_type=jnp.float32)
    h_ref[...] = h
    _node_tail(h, as_ref, ad_ref, asrc_ref, adst_ref, M_ref, mx_ref, i)


def _tcb_body(U_ref, den_ref, b_ref, W_ref, as_ref, ad_ref,
              h_ref, asrc_ref, adst_ref, M_ref, mx_ref):
    i = pl.program_id(0)
    u = U_ref[0] + U_ref[1]
    den = den_ref[0] + den_ref[1]
    out = jnp.maximum(u / (den + EPS) + b_ref[...], 0.0)
    h = jnp.dot(out, W_ref[...], preferred_element_type=jnp.float32)
    h_ref[...] = h
    _node_tail(h, as_ref, ad_ref, asrc_ref, adst_ref, M_ref, mx_ref, i)


def _tcc_body(U_ref, den_ref, b_ref, batch_ref, Wf_ref, bf_ref,
              y_ref, acc_ref):
    i = pl.program_id(0)
    u = U_ref[0] + U_ref[1]
    den = den_ref[0] + den_ref[1]
    out = jnp.maximum(u / (den + EPS) + b_ref[...], 0.0)
    gids = lax.broadcasted_iota(jnp.int32, (NBLK, G), 1)
    mask = (batch_ref[...] == gids).astype(jnp.float32)

    @pl.when(i == 0)
    def _():
        acc_ref[...] = jnp.zeros_like(acc_ref)

    acc_ref[...] += lax.dot_general(mask, out, (((0,), (0,)), ((), ())),
                                    preferred_element_type=jnp.float32)

    @pl.when(i == pl.num_programs(0) - 1)
    def _():
        y_ref[...] = jnp.dot(acc_ref[...], Wf_ref[...],
                             preferred_element_type=jnp.float32) + bf_ref[0]


def _tc_node_first(x, W, a_s, a_d):
    return pl.pallas_call(
        _tca_body,
        grid=(N // NBLK,),
        in_specs=[
            pl.BlockSpec((NBLK, H), lambda i: (i, 0)),
            pl.BlockSpec((H, H), lambda i: (0, 0)),
            pl.BlockSpec((1, H), lambda i: (0, 0)),
            pl.BlockSpec((1, H), lambda i: (0, 0)),
        ],
        out_specs=[
            pl.BlockSpec((NBLK, H), lambda i: (i, 0)),
            pl.BlockSpec((NBLK, 1), lambda i: (i, 0)),
            pl.BlockSpec((NBLK, 1), lambda i: (i, 0)),
            pl.BlockSpec(memory_space=pltpu.SMEM),
        ],
        out_shape=[
            jax.ShapeDtypeStruct((N, H), jnp.float32),
            jax.ShapeDtypeStruct((N, 1), jnp.float32),
            jax.ShapeDtypeStruct((N, 1), jnp.float32),
            jax.ShapeDtypeStruct((1,), jnp.float32),
        ],
        scratch_shapes=[pltpu.SMEM((2,), jnp.float32)],
    )(x, W, a_s.reshape(1, H), a_d.reshape(1, H))


def _tc_node_mid(U, den, b, W, a_s, a_d):
    return pl.pallas_call(
        _tcb_body,
        grid=(N // NBLK,),
        in_specs=[
            pl.BlockSpec((2, NBLK, H), lambda i: (0, i, 0)),
            pl.BlockSpec((2, NBLK, 1), lambda i: (0, i, 0)),
            pl.BlockSpec((1, H), lambda i: (0, 0)),
            pl.BlockSpec((H, H), lambda i: (0, 0)),
            pl.BlockSpec((1, H), lambda i: (0, 0)),
            pl.BlockSpec((1, H), lambda i: (0, 0)),
        ],
        out_specs=[
            pl.BlockSpec((NBLK, H), lambda i: (i, 0)),
            pl.BlockSpec((NBLK, 1), lambda i: (i, 0)),
            pl.BlockSpec((NBLK, 1), lambda i: (i, 0)),
            pl.BlockSpec(memory_space=pltpu.SMEM),
        ],
        out_shape=[
            jax.ShapeDtypeStruct((N, H), jnp.float32),
            jax.ShapeDtypeStruct((N, 1), jnp.float32),
            jax.ShapeDtypeStruct((N, 1), jnp.float32),
            jax.ShapeDtypeStruct((1,), jnp.float32),
        ],
        scratch_shapes=[pltpu.SMEM((2,), jnp.float32)],
    )(U, den.reshape(2, N, 1), b.reshape(1, H), W,
      a_s.reshape(1, H), a_d.reshape(1, H))


def _tc_pool(U, den, b, batch, Wf, bf):
    return pl.pallas_call(
        _tcc_body,
        grid=(N // NBLK,),
        in_specs=[
            pl.BlockSpec((2, NBLK, H), lambda i: (0, i, 0)),
            pl.BlockSpec((2, NBLK, 1), lambda i: (0, i, 0)),
            pl.BlockSpec((1, H), lambda i: (0, 0)),
            pl.BlockSpec((NBLK, 1), lambda i: (i, 0)),
            pl.BlockSpec((H, 1), lambda i: (0, 0)),
            pl.BlockSpec(memory_space=pltpu.SMEM),
        ],
        out_specs=pl.BlockSpec((G, 1), lambda i: (0, 0)),
        out_shape=jax.ShapeDtypeStruct((G, 1), jnp.float32),
        scratch_shapes=[pltpu.VMEM((G, H), jnp.float32)],
    )(U, den.reshape(2, N, 1), b.reshape(1, H),
      batch.reshape(N, 1), Wf, bf)


@functools.partial(
    pl.kernel,
    out_type=(jax.ShapeDtypeStruct((2, N, H), jnp.float32),
              jax.ShapeDtypeStruct((2, N), jnp.float32)),
    mesh=plsc.VectorSubcoreMesh(core_axis_name="c", subcore_axis_name="s"),
    compiler_params=pltpu.CompilerParams(needs_layout_passes=False,
                                         use_tc_tiling_on_sc=False),
    scratch_types=[
        pltpu.VMEM_SHARED((N, H), jnp.float32),
        pltpu.VMEM_SHARED((N,), jnp.float32),
        pltpu.VMEM((2 * BB,), jnp.int32),
        pltpu.VMEM((2 * BB,), jnp.int32),
        pltpu.VMEM((BB,), jnp.int32),
        pltpu.VMEM((BB,), jnp.int32),
        pltpu.VMEM((BB,), jnp.float32),
        pltpu.VMEM((BB,), jnp.float32),
        pltpu.VMEM((BB,), jnp.float32),
        pltpu.VMEM((BB,), jnp.float32),
        pltpu.VMEM((BB,), jnp.float32),
        pltpu.VMEM((BB,), jnp.float32),
        pltpu.VMEM((BB, H // 2), jnp.int32),
        pltpu.VMEM((BB, H // 2), jnp.int32),
        pltpu.VMEM((BB, H), jnp.float32),
        pltpu.VMEM((BB, H), jnp.float32),
        pltpu.VMEM((16,), jnp.float32),
        pltpu.SemaphoreType.DMA,
        pltpu.SemaphoreType.DMA,
        pltpu.SemaphoreType.DMA,
        pltpu.SemaphoreType.DMA,
        pltpu.SemaphoreType.DMA,
        pltpu.SemaphoreType.DMA,
        pltpu.SemaphoreType.DMA,
        pltpu.SemaphoreType.DMA,
    ],
)
def _sc_edge(h_hbm, asrc_hbm, adst_hbm, m_hbm,
             sd2_hbm, zr_hbm, zd_hbm,
             U_out, den_out,
             U_sh, den_sh,
             sdb0, sdb1, dstb0, dstb1, exb0, exb1, asb0, asb1, adb0, adb1,
             rbf0, rbf1, rf0, rf1, m_v,
             gs0, gs1, al0, al1, us0, us1, dn0, dn1):
    cid = lax.axis_index("c")
    sid = lax.axis_index("s")
    wid = sid * 2 + cid

    @pl.when(sid < 15)
    def _():
        off = pl.multiple_of(sid * RS, 8)
        pltpu.sync_copy(zr_hbm.at[pl.ds(off, RS)], U_sh.at[pl.ds(off, RS)])

    @pl.when(sid == 15)
    def _():
        pltpu.sync_copy(zr_hbm.at[pl.ds(15 * RS, RSL)],
                        U_sh.at[pl.ds(15 * RS, RSL)])

    @pl.when(sid == 0)
    def _():
        pltpu.sync_copy(zd_hbm, den_sh)

    pltpu.sync_copy(m_hbm, m_v)
    plsc.subcore_barrier()

    m = m_v[...]
    bufs = ((sdb0, dstb0, exb0, asb0, adb0, rbf0, rf0, gs0, al0, us0, dn0),
            (sdb1, dstb1, exb1, asb1, adb1, rbf1, rf1, gs1, al1, us1, dn1))

    def _prefetch(bn, buf, do_waits):
        sdb, dstb, exb, asb, adb, rbf, rf, gs, al, us, dn = buf

        @pl.when(do_waits)
        def _():
            pltpu.make_async_copy(rf, U_sh.at[dstb], us).wait()
            pltpu.make_async_copy(exb, den_sh.at[dstb], dn).wait()

        pltpu.sync_copy(sd2_hbm.at[wid * NBATCH + bn], sdb)
        for k in range(BB // 16):
            dstb[pl.ds(k * 16, 16)] = sdb[pl.ds(BB + k * 16, 16)]
        pltpu.async_copy(h_hbm.at[sdb.at[pl.ds(0, BB)]], rbf, gs)
        pltpu.async_copy(asrc_hbm.at[sdb.at[pl.ds(0, BB)]], asb, al)
        pltpu.async_copy(adst_hbm.at[dstb], adb, al)

    def _process(buf):
        sdb, dstb, exb, asb, adb, rbf, rf, gs, al, us, dn = buf
        pltpu.make_async_copy(asrc_hbm.at[sdb.at[pl.ds(0, BB)]], asb, al).wait()
        pltpu.make_async_copy(adst_hbm.at[dstb], adb, al).wait()

        @plsc.parallel_loop(0, BB // 16)
        def _(g):
            a = asb[pl.ds(g * 16, 16)] + adb[pl.ds(g * 16, 16)]
            a = jnp.maximum(a, a * NEG)
            exb[pl.ds(g * 16, 16)] = jnp.exp(a - m)

        pltpu.async_copy(exb, den_sh.at[dstb], dn, add=True)
        pltpu.make_async_copy(h_hbm.at[sdb.at[pl.ds(0, BB)]], rbf, gs).wait()

        @plsc.parallel_loop(0, BB, unroll=4)
        def _(e):
            eidx = jnp.full((16,), e, jnp.int32)
            ev = plsc.load_gather(exb, [eidx])
            for k in range(H // 32):
                x = rbf[e, pl.ds(k * 16, 16)]
                lo = plsc.bitcast(x << 16, jnp.float32) * ev
                hi = plsc.bitcast(x & jnp.int32(-65536), jnp.float32) * ev
                rf[e, pl.ds(k * 16, 16)] = lo
                rf[e, pl.ds(64 + k * 16, 16)] = hi

        pltpu.async_copy(rf, U_sh.at[dstb], us, add=True)

    _prefetch(0, bufs[0], False)

    @pl.loop(0, NBATCH)
    def _(b):
        even = b % 2 == 0
        more = b + 1 < NBATCH

        @pl.when(jnp.logical_and(even, more))
        def _():
            _prefetch(b + 1, bufs[1], b >= 1)

        @pl.when(jnp.logical_and(jnp.logical_not(even), more))
        def _():
            _prefetch(b + 1, bufs[0], b >= 1)

        @pl.when(even)
        def _():
            _process(bufs[0])

        @pl.when(jnp.logical_not(even))
        def _():
            _process(bufs[1])

    pltpu.make_async_copy(rf0, U_sh.at[dstb0], us0).wait()
    pltpu.make_async_copy(exb0, den_sh.at[dstb0], dn0).wait()
    pltpu.make_async_copy(rf1, U_sh.at[dstb1], us1).wait()
    pltpu.make_async_copy(exb1, den_sh.at[dstb1], dn1).wait()

    plsc.subcore_barrier()

    @pl.when(sid < 15)
    def _():
        off = pl.multiple_of(sid * RS, 8)
        pltpu.sync_copy(U_sh.at[pl.ds(off, RS)],
                        U_out.at[cid, pl.ds(off, RS)])

    @pl.when(sid == 15)
    def _():
        pltpu.sync_copy(U_sh.at[pl.ds(15 * RS, RSL)],
                        U_out.at[cid, pl.ds(15 * RS, RSL)])

    @pl.when(sid == 0)
    def _():
        pltpu.sync_copy(den_sh, den_out.at[cid])


def kernel(x, edge_index, batch, dense_edge_idx, W1, a_src1, a_dst1, b1,
           W2, a_src2, a_dst2, b2, Wf, bf):
    src2 = edge_index[0].reshape(E // BB, BB)
    dst2 = edge_index[1].reshape(E // BB, BB)
    sd2 = jnp.concatenate([src2, dst2], axis=1)
    zr = jnp.zeros((N, H), jnp.float32)
    zd = jnp.zeros((N,), jnp.float32)

    def _pack(h):
        hb = h.astype(jnp.bfloat16).reshape(N, H // 2, 2)
        return lax.bitcast_convert_type(hb, jnp.int32)

    perm = jnp.concatenate([jnp.arange(0, H, 2), jnp.arange(1, H, 2)])

    h1, asrc1, adst1, M1 = _tc_node_first(x, W1, a_src1, a_dst1)
    U1, den1 = _sc_edge(_pack(h1), asrc1.reshape(N), adst1.reshape(N),
                        jnp.broadcast_to(M1, (16,)), sd2, zr, zd)
    h2, asrc2, adst2, M2 = _tc_node_mid(U1, den1, b1[perm], W2[perm, :],
                                        a_src2, a_dst2)
    U2, den2 = _sc_edge(_pack(h2), asrc2.reshape(N), adst2.reshape(N),
                        jnp.broadcast_to(M2, (16,)), sd2, zr, zd)
    y = _tc_pool(U2, den2, b2[perm], batch, Wf[perm, :], bf.reshape(1))
    return y[:, 0]


# speedup vs baseline: 1.1654x; 1.1307x over previous
import functools

import jax
import jax.numpy as jnp
from jax import lax
from jax.experimental import pallas as pl
from jax.experimental.pallas import tpu as pltpu
from jax.experimental.pallas import tpu_sc as plsc

N = 10000
E = 320000
H = 128
G = 64
NEG = 0.2
EPS = 1e-30

NBLK = 2000
NW = 32
EC = E // NW
BB = 80
NBATCH = EC // BB
RS = 632
RSL = N - 15 * RS


def _node_tail(h, as_ref, ad_ref, asrc_ref, adst_ref, M_ref, mx_ref, i):
    a_s = jnp.sum(h * as_ref[...], axis=1, keepdims=True)
    a_d = jnp.sum(h * ad_ref[...], axis=1, keepdims=True)
    asrc_ref[...] = a_s
    adst_ref[...] = a_d

    @pl.when(i == 0)
    def _():
        mx_ref[0] = jnp.max(a_s)
        mx_ref[1] = jnp.max(a_d)

    mx_ref[0] = jnp.maximum(mx_ref[0], jnp.max(a_s))
    mx_ref[1] = jnp.maximum(mx_ref[1], jnp.max(a_d))

    @pl.when(i == pl.num_programs(0) - 1)
    def _():
        M_ref[0] = jnp.maximum(mx_ref[0] + mx_ref[1], 0.0)


def _tca_body(x_ref, W_ref, as_ref, ad_ref,
              h_ref, asrc_ref, adst_ref, M_ref, mx_ref):
    i = pl.program_id(0)
    h = jnp.dot(x_ref[...], W_ref[...], preferred_element_type=jnp.float32)
    h_ref[...] = h
    _node_tail(h, as_ref, ad_ref, asrc_ref, adst_ref, M_ref, mx_ref, i)


def _tcb_body(U_ref, den_ref, b_ref, W_ref, as_ref, ad_ref,
              h_ref, asrc_ref, adst_ref, M_ref, mx_ref):
    i = pl.program_id(0)
    u = U_ref[0] + U_ref[1]
    den = den_ref[0] + den_ref[1]
    out = jnp.maximum(u / (den + EPS) + b_ref[...], 0.0)
    h = jnp.dot(out, W_ref[...], preferred_element_type=jnp.float32)
    h_ref[...] = h
    _node_tail(h, as_ref, ad_ref, asrc_ref, adst_ref, M_ref, mx_ref, i)


def _tcc_body(U_ref, den_ref, b_ref, batch_ref, Wf_ref, bf_ref,
              y_ref, acc_ref):
    i = pl.program_id(0)
    u = U_ref[0] + U_ref[1]
    den = den_ref[0] + den_ref[1]
    out = jnp.maximum(u / (den + EPS) + b_ref[...], 0.0)
    gids = lax.broadcasted_iota(jnp.int32, (NBLK, G), 1)
    mask = (batch_ref[...] == gids).astype(jnp.float32)

    @pl.when(i == 0)
    def _():
        acc_ref[...] = jnp.zeros_like(acc_ref)

    acc_ref[...] += lax.dot_general(mask, out, (((0,), (0,)), ((), ())),
                                    preferred_element_type=jnp.float32)

    @pl.when(i == pl.num_programs(0) - 1)
    def _():
        y_ref[...] = jnp.dot(acc_ref[...], Wf_ref[...],
                             preferred_element_type=jnp.float32) + bf_ref[0]


def _tc_node_first(x, W, a_s, a_d):
    return pl.pallas_call(
        _tca_body,
        grid=(N // NBLK,),
        in_specs=[
            pl.BlockSpec((NBLK, H), lambda i: (i, 0)),
            pl.BlockSpec((H, H), lambda i: (0, 0)),
            pl.BlockSpec((1, H), lambda i: (0, 0)),
            pl.BlockSpec((1, H), lambda i: (0, 0)),
        ],
        out_specs=[
            pl.BlockSpec((NBLK, H), lambda i: (i, 0)),
            pl.BlockSpec((NBLK, 1), lambda i: (i, 0)),
            pl.BlockSpec((NBLK, 1), lambda i: (i, 0)),
            pl.BlockSpec(memory_space=pltpu.SMEM),
        ],
        out_shape=[
            jax.ShapeDtypeStruct((N, H), jnp.float32),
            jax.ShapeDtypeStruct((N, 1), jnp.float32),
            jax.ShapeDtypeStruct((N, 1), jnp.float32),
            jax.ShapeDtypeStruct((1,), jnp.float32),
        ],
        scratch_shapes=[pltpu.SMEM((2,), jnp.float32)],
    )(x, W, a_s.reshape(1, H), a_d.reshape(1, H))


def _tc_node_mid(U, den, b, W, a_s, a_d):
    return pl.pallas_call(
        _tcb_body,
        grid=(N // NBLK,),
        in_specs=[
            pl.BlockSpec((2, NBLK, H), lambda i: (0, i, 0)),
            pl.BlockSpec((2, NBLK, 1), lambda i: (0, i, 0)),
            pl.BlockSpec((1, H), lambda i: (0, 0)),
            pl.BlockSpec((H, H), lambda i: (0, 0)),
            pl.BlockSpec((1, H), lambda i: (0, 0)),
            pl.BlockSpec((1, H), lambda i: (0, 0)),
        ],
        out_specs=[
            pl.BlockSpec((NBLK, H), lambda i: (i, 0)),
            pl.BlockSpec((NBLK, 1), lambda i: (i, 0)),
            pl.BlockSpec((NBLK, 1), lambda i: (i, 0)),
            pl.BlockSpec(memory_space=pltpu.SMEM),
        ],
        out_shape=[
            jax.ShapeDtypeStruct((N, H), jnp.float32),
            jax.ShapeDtypeStruct((N, 1), jnp.float32),
            jax.ShapeDtypeStruct((N, 1), jnp.float32),
            jax.ShapeDtypeStruct((1,), jnp.float32),
        ],
        scratch_shapes=[pltpu.SMEM((2,), jnp.float32)],
    )(U, den.reshape(2, N, 1), b.reshape(1, H), W,
      a_s.reshape(1, H), a_d.reshape(1, H))


def _tc_pool(U, den, b, batch, Wf, bf):
    return pl.pallas_call(
        _tcc_body,
        grid=(N // NBLK,),
        in_specs=[
            pl.BlockSpec((2, NBLK, H), lambda i: (0, i, 0)),
            pl.BlockSpec((2, NBLK, 1), lambda i: (0, i, 0)),
            pl.BlockSpec((1, H), lambda i: (0, 0)),
            pl.BlockSpec((NBLK, 1), lambda i: (i, 0)),
            pl.BlockSpec((H, 1), lambda i: (0, 0)),
            pl.BlockSpec(memory_space=pltpu.SMEM),
        ],
        out_specs=pl.BlockSpec((G, 1), lambda i: (0, 0)),
        out_shape=jax.ShapeDtypeStruct((G, 1), jnp.float32),
        scratch_shapes=[pltpu.VMEM((G, H), jnp.float32)],
    )(U, den.reshape(2, N, 1), b.reshape(1, H),
      batch.reshape(N, 1), Wf, bf)


@functools.partial(
    pl.kernel,
    out_type=(jax.ShapeDtypeStruct((2, N, H), jnp.float32),
              jax.ShapeDtypeStruct((2, N), jnp.float32)),
    mesh=plsc.VectorSubcoreMesh(core_axis_name="c", subcore_axis_name="s"),
    compiler_params=pltpu.CompilerParams(needs_layout_passes=False),
    scratch_types=[
        pltpu.VMEM_SHARED((N, H), jnp.float32),
        pltpu.VMEM_SHARED((N,), jnp.float32),
        pltpu.VMEM((N,), jnp.float32),
        pltpu.VMEM((N,), jnp.float32),
        pltpu.VMEM((2 * BB,), jnp.int32),
        pltpu.VMEM((2 * BB,), jnp.int32),
        pltpu.VMEM((BB,), jnp.int32),
        pltpu.VMEM((BB,), jnp.int32),
        pltpu.VMEM((BB,), jnp.float32),
        pltpu.VMEM((BB,), jnp.float32),
        pltpu.VMEM((BB, H), jnp.float32),
        pltpu.VMEM((BB, H), jnp.float32),
        pltpu.VMEM((16,), jnp.float32),
        pltpu.SemaphoreType.DMA,
        pltpu.SemaphoreType.DMA,
        pltpu.SemaphoreType.DMA,
        pltpu.SemaphoreType.DMA,
        pltpu.SemaphoreType.DMA,
        pltpu.SemaphoreType.DMA,
    ],
)
def _sc_edge(h_hbm, asrc_hbm, adst_hbm, m_hbm,
             sd2_hbm, zr_hbm, zd_hbm,
             U_out, den_out,
             U_sh, den_sh, asv, adv,
             sdb0, sdb1, dstb0, dstb1, exb0, exb1, rows0, rows1, m_v,
             gs0, gs1, us0, us1, dn0, dn1):
    cid = lax.axis_index("c")
    sid = lax.axis_index("s")
    wid = sid * 2 + cid

    @pl.when(sid < 15)
    def _():
        off = pl.multiple_of(sid * RS, 8)
        pltpu.sync_copy(zr_hbm.at[pl.ds(off, RS)], U_sh.at[pl.ds(off, RS)])

    @pl.when(sid == 15)
    def _():
        pltpu.sync_copy(zr_hbm.at[pl.ds(15 * RS, RSL)],
                        U_sh.at[pl.ds(15 * RS, RSL)])

    @pl.when(sid == 0)
    def _():
        pltpu.sync_copy(zd_hbm, den_sh)

    pltpu.sync_copy(asrc_hbm, asv)
    pltpu.sync_copy(adst_hbm, adv)
    pltpu.sync_copy(m_hbm, m_v)
    plsc.subcore_barrier()

    m = m_v[...]
    bufs = ((sdb0, dstb0, exb0, rows0, gs0, us0, dn0),
            (sdb1, dstb1, exb1, rows1, gs1, us1, dn1))

    def _prefetch(bn, buf, do_waits):
        sdb, dstb, exb, rows, gs, us, dn = buf

        @pl.when(do_waits)
        def _():
            pltpu.make_async_copy(rows, U_sh.at[dstb], us).wait()
            pltpu.make_async_copy(exb, den_sh.at[dstb], dn).wait()

        pltpu.sync_copy(sd2_hbm.at[wid * NBATCH + bn], sdb)
        for k in range(BB // 16):
            dstb[pl.ds(k * 16, 16)] = sdb[pl.ds(BB + k * 16, 16)]
        pltpu.async_copy(h_hbm.at[sdb.at[pl.ds(0, BB)]], rows, gs)

        @plsc.parallel_loop(0, BB // 16)
        def _(g):
            sidx = sdb[pl.ds(g * 16, 16)]
            didx = dstb[pl.ds(g * 16, 16)]
            a = plsc.load_gather(asv, [sidx]) + plsc.load_gather(adv, [didx])
            a = jnp.maximum(a, a * NEG)
            exb[pl.ds(g * 16, 16)] = jnp.exp(a - m)

        pltpu.async_copy(exb, den_sh.at[dstb], dn, add=True)

    def _process(buf):
        sdb, dstb, exb, rows, gs, us, dn = buf
        pltpu.make_async_copy(h_hbm.at[sdb.at[pl.ds(0, BB)]], rows, gs).wait()

        @plsc.parallel_loop(0, BB, unroll=8)
        def _(e):
            ev = plsc.load_gather(exb, [jnp.full((16,), e, jnp.int32)])
            for k in range(H // 16):
                rows[e, pl.ds(k * 16, 16)] = rows[e, pl.ds(k * 16, 16)] * ev

        pltpu.async_copy(rows, U_sh.at[dstb], us, add=True)

    _prefetch(0, bufs[0], False)

    @pl.loop(0, NBATCH)
    def _(b):
        even = b % 2 == 0
        more = b + 1 < NBATCH

        @pl.when(jnp.logical_and(even, more))
        def _():
            _prefetch(b + 1, bufs[1], b >= 1)

        @pl.when(jnp.logical_and(jnp.logical_not(even), more))
        def _():
            _prefetch(b + 1, bufs[0], b >= 1)

        @pl.when(even)
        def _():
            _process(bufs[0])

        @pl.when(jnp.logical_not(even))
        def _():
            _process(bufs[1])

    pltpu.make_async_copy(rows0, U_sh.at[dstb0], us0).wait()
    pltpu.make_async_copy(exb0, den_sh.at[dstb0], dn0).wait()
    pltpu.make_async_copy(rows1, U_sh.at[dstb1], us1).wait()
    pltpu.make_async_copy(exb1, den_sh.at[dstb1], dn1).wait()

    plsc.subcore_barrier()

    @pl.when(sid < 15)
    def _():
        off = pl.multiple_of(sid * RS, 8)
        pltpu.sync_copy(U_sh.at[pl.ds(off, RS)],
                        U_out.at[cid, pl.ds(off, RS)])

    @pl.when(sid == 15)
    def _():
        pltpu.sync_copy(U_sh.at[pl.ds(15 * RS, RSL)],
                        U_out.at[cid, pl.ds(15 * RS, RSL)])

    @pl.when(sid == 0)
    def _():
        pltpu.sync_copy(den_sh, den_out.at[cid])


def kernel(x, edge_index, batch, dense_edge_idx, W1, a_src1, a_dst1, b1,
           W2, a_src2, a_dst2, b2, Wf, bf):
    src2 = edge_index[0].reshape(E // BB, BB)
    dst2 = edge_index[1].reshape(E // BB, BB)
    sd2 = jnp.concatenate([src2, dst2], axis=1)
    zr = jnp.zeros((N, H), jnp.float32)
    zd = jnp.zeros((N,), jnp.float32)

    h1, asrc1, adst1, M1 = _tc_node_first(x, W1, a_src1, a_dst1)
    U1, den1 = _sc_edge(h1, asrc1.reshape(N), adst1.reshape(N),
                        jnp.broadcast_to(M1, (16,)), sd2, zr, zd)
    h2, asrc2, adst2, M2 = _tc_node_mid(U1, den1, b1, W2, a_src2, a_dst2)
    U2, den2 = _sc_edge(h2, asrc2.reshape(N), adst2.reshape(N),
                        jnp.broadcast_to(M2, (16,)), sd2, zr, zd)
    y = _tc_pool(U2, den2, b2, batch, Wf, bf.reshape(1))
    return y[:, 0]
